# Initial kernel scaffold; baseline (speedup 1.0000x reference)
import jax
import jax.numpy as jnp
from jax.experimental import pallas as pl


def kernel(accumulated):
    raise NotImplementedError("write your pallas kernel here")


# speedup vs baseline: 2.2787x; 2.2787x over previous
import functools

import jax
import jax.numpy as jnp
from jax import lax
from jax.experimental import pallas as pl
from jax.experimental.pallas import tpu as pltpu
from jax.experimental.pallas import tpu_sc as plsc

B = 128
N = 32768
K = 64
L = 16
NV = N // L
NC = 2
NS = 16
NW = NC * NS
ROWS_PER_W = B // NW
CMAX = 4096
NEG = float("-inf")
BIG = 1 << 30


def _splat_f(x):
    return jnp.full((L,), x, dtype=jnp.float32)


def _splat_i(x):
    return jnp.full((L,), x, dtype=jnp.int32)


def _select_topk(val_load, val_kill, idx_of, nvregs, ovbuf, oibuf, lane0, iota):
    def k_body(k, _):
        def scan(j, carry):
            lmax, lpos = carry
            x = val_load(j)
            gt = x > lmax
            lmax = jnp.maximum(lmax, x)
            lpos = jnp.where(gt, _splat_i(j * L) + iota, lpos)
            return (lmax, lpos)

        lmax, lpos = lax.fori_loop(
            0, nvregs, scan, (_splat_f(NEG), _splat_i(0)))
        m = jnp.max(lmax)
        msp = _splat_f(m)
        cand = jnp.where(lmax == msp, lpos, BIG)
        found = jnp.min(cand)
        fsp = _splat_i(found)
        plsc.store_scatter(ovbuf, [_splat_i(k)], msp, mask=lane0)
        plsc.store_scatter(oibuf, [_splat_i(k)], idx_of(fsp), mask=lane0)
        val_kill(fsp)
        return 0

    lax.fori_loop(0, K, k_body, 0)


def _topk_body(acc_hbm, outv_hbm, outi_hbm,
               rowbuf0, rowbuf1, cval, cidx, ovbuf, oibuf, sem0, sem1):
    wid = lax.axis_index("s") * NC + lax.axis_index("c")
    base_row = wid * ROWS_PER_W
    sems = (sem0, sem1)
    iota = lax.broadcasted_iota(jnp.int32, (L,), 0)
    lane0 = iota == 0
    neg16 = _splat_f(NEG)

    bufs = (rowbuf0, rowbuf1)
    handles = [None, None]
    handles[0] = pltpu.async_copy(acc_hbm.at[base_row], bufs[0], sems[0])
    for r in range(ROWS_PER_W):
        cur = r % 2
        nxt = (r + 1) % 2
        if r + 1 < ROWS_PER_W:
            handles[nxt] = pltpu.async_copy(
                acc_hbm.at[base_row + (r + 1)], bufs[nxt], sems[nxt])
        handles[cur].wait()
        row = bufs[cur]

        QB = NV // 4

        def p1(i, accs):
            a0, a1, a2, a3 = accs
            a0 = jnp.maximum(a0, row[pl.ds(i * L, L)])
            a1 = jnp.maximum(a1, row[pl.ds((QB + i) * L, L)])
            a2 = jnp.maximum(a2, row[pl.ds((2 * QB + i) * L, L)])
            a3 = jnp.maximum(a3, row[pl.ds((3 * QB + i) * L, L)])
            return (a0, a1, a2, a3)

        a0, a1, a2, a3 = lax.fori_loop(0, QB, p1, (neg16, neg16, neg16, neg16))
        thr = jnp.min(jnp.minimum(jnp.minimum(a0, a1), jnp.minimum(a2, a3)))
        thr_s = _splat_f(thr)

        def p2(i, carry):
            cnt, ivec = carry
            x = row[pl.ds(i * L, L)]
            msk = x >= thr_s
            c = jnp.max(plsc.all_reduce_population_count(msk))

            def do_write(cnt):
                off = jnp.minimum(cnt, CMAX)
                plsc.store_compressed(cval.at[pl.ds(off, L)], x, mask=msk)
                plsc.store_compressed(cidx.at[pl.ds(off, L)], ivec, mask=msk)
                return cnt + c

            cnt = lax.cond(c > 0, do_write, lambda z: z, cnt)
            return (cnt, ivec + L)

        cnt, _ = lax.fori_loop(0, NV, p2, (jnp.int32(0), iota))
        cval[pl.ds(jnp.minimum(cnt, CMAX), L)] = neg16

        def normal(_):
            _select_topk(
                val_load=lambda j: cval[pl.ds(j * L, L)],
                val_kill=lambda fsp: plsc.store_scatter(
                    cval, [fsp], neg16, mask=lane0),
                idx_of=lambda fsp: plsc.load_gather(cidx, [fsp]),
                nvregs=(cnt + L - 1) // L,
                ovbuf=ovbuf, oibuf=oibuf, lane0=lane0, iota=iota)
            return 0

        def fallback(_):
            _select_topk(
                val_load=lambda j: row[pl.ds(j * L, L)],
                val_kill=lambda fsp: plsc.store_scatter(
                    row, [fsp], neg16, mask=lane0),
                idx_of=lambda fsp: fsp,
                nvregs=NV,
                ovbuf=ovbuf, oibuf=oibuf, lane0=lane0, iota=iota)
            return 0

        lax.cond(cnt <= CMAX, normal, fallback, 0)

        pltpu.sync_copy(ovbuf, outv_hbm.at[base_row + r])
        pltpu.sync_copy(oibuf, outi_hbm.at[base_row + r])


@functools.lru_cache(maxsize=1)
def _topk_call():
    return functools.partial(
        pl.kernel,
        out_type=[
            jax.ShapeDtypeStruct((B, K), jnp.float32),
            jax.ShapeDtypeStruct((B, K), jnp.int32),
        ],
        mesh=plsc.VectorSubcoreMesh(core_axis_name="c", subcore_axis_name="s"),
        compiler_params=pltpu.CompilerParams(needs_layout_passes=False),
        scratch_types=[
            pltpu.VMEM((N,), jnp.float32),
            pltpu.VMEM((N,), jnp.float32),
            pltpu.VMEM((CMAX + L,), jnp.float32),
            pltpu.VMEM((CMAX + L,), jnp.int32),
            pltpu.VMEM((K,), jnp.float32),
            pltpu.VMEM((K,), jnp.int32),
            pltpu.SemaphoreType.DMA,
            pltpu.SemaphoreType.DMA,
        ],
    )(_topk_body)


def kernel(accumulated):
    topk_vals, topk_idx = _topk_call()(accumulated)
    return (topk_vals, topk_idx, accumulated)


# speedup vs baseline: 5.6843x; 2.4945x over previous
import functools

import jax
import jax.numpy as jnp
from jax import lax
from jax.experimental import pallas as pl
from jax.experimental.pallas import tpu as pltpu
from jax.experimental.pallas import tpu_sc as plsc

B = 128
N = 32768
K = 64
L = 16
NV = N // L
NC = 2
NS = 16
NW = NC * NS
ROWS_PER_W = B // NW
CMAX = 4096
NEG = float("-inf")
BIG = 1 << 30


def _splat_f(x):
    return jnp.full((L,), x, dtype=jnp.float32)


def _splat_i(x):
    return jnp.full((L,), x, dtype=jnp.int32)


SU = 4


def _select_topk(val_load, val_kill, idx_of, ngroups, ovbuf, oibuf, lane0, iota):
    def k_body(k, _):
        def scan(jg, carry):
            lmax, lpos = carry
            for u in range(SU):
                j = jg * SU + u
                x = val_load(j)
                gt = x > lmax
                lmax = jnp.maximum(lmax, x)
                lpos = jnp.where(gt, _splat_i(j * L) + iota, lpos)
            return (lmax, lpos)

        lmax, lpos = lax.fori_loop(
            0, ngroups, scan, (_splat_f(NEG), _splat_i(0)))
        m = jnp.max(lmax)
        msp = _splat_f(m)
        cand = jnp.where(lmax == msp, lpos, BIG)
        found = jnp.min(cand)
        fsp = _splat_i(found)
        plsc.store_scatter(ovbuf, [_splat_i(k)], msp, mask=lane0)
        plsc.store_scatter(oibuf, [_splat_i(k)], idx_of(fsp), mask=lane0)
        val_kill(fsp)
        return 0

    lax.fori_loop(0, K, k_body, 0)


def _topk_body(acc_hbm, outv_hbm, outi_hbm,
               rowbuf0, rowbuf1, cval, cidx, ovbuf, oibuf, sem0, sem1):
    wid = lax.axis_index("s") * NC + lax.axis_index("c")
    base_row = wid * ROWS_PER_W
    sems = (sem0, sem1)
    iota = lax.broadcasted_iota(jnp.int32, (L,), 0)
    lane0 = iota == 0
    neg16 = _splat_f(NEG)

    bufs = (rowbuf0, rowbuf1)
    handles = [None, None]
    handles[0] = pltpu.async_copy(acc_hbm.at[base_row], bufs[0], sems[0])
    for r in range(ROWS_PER_W):
        cur = r % 2
        nxt = (r + 1) % 2
        if r + 1 < ROWS_PER_W:
            handles[nxt] = pltpu.async_copy(
                acc_hbm.at[base_row + (r + 1)], bufs[nxt], sems[nxt])
        handles[cur].wait()
        row = bufs[cur]

        QB = NV // 4
        U1 = 4

        def p1(i, accs):
            a0, a1, a2, a3 = accs
            for u in range(U1):
                a0 = jnp.maximum(a0, row[pl.ds((i * U1 + u) * L, L)])
                a1 = jnp.maximum(a1, row[pl.ds((QB + i * U1 + u) * L, L)])
                a2 = jnp.maximum(a2, row[pl.ds((2 * QB + i * U1 + u) * L, L)])
                a3 = jnp.maximum(a3, row[pl.ds((3 * QB + i * U1 + u) * L, L)])
            return (a0, a1, a2, a3)

        a0, a1, a2, a3 = lax.fori_loop(0, QB // U1, p1,
                                       (neg16, neg16, neg16, neg16))
        thr = jnp.min(jnp.minimum(jnp.minimum(a0, a1), jnp.minimum(a2, a3)))
        thr_s = _splat_f(thr)

        G = 8
        lim_s = _splat_i(CMAX + L)

        def p2(g, cntv):
            base = g * (G * L)
            xs = [row[pl.ds(base + k * L, L)] for k in range(G)]
            m01 = jnp.maximum(xs[0], xs[1])
            m23 = jnp.maximum(xs[2], xs[3])
            m45 = jnp.maximum(xs[4], xs[5])
            m67 = jnp.maximum(xs[6], xs[7])
            mx = jnp.maximum(jnp.maximum(m01, m23), jnp.maximum(m45, m67))
            has = jnp.any(mx >= thr_s)

            def taken(cntv):
                for k in range(G):
                    msk = xs[k] >= thr_s
                    pfx = plsc.cumsum(msk.astype(jnp.int32))
                    tgt = cntv + pfx - 1
                    ok = msk & (tgt < lim_s)
                    plsc.store_scatter(cval, [tgt], xs[k], mask=ok)
                    plsc.store_scatter(
                        cidx, [tgt], iota + (base + k * L), mask=ok)
                    cntv = cntv + plsc.all_reduce_population_count(msk)
                return cntv

            return lax.cond(has, taken, lambda z: z, cntv)

        cntv = lax.fori_loop(0, NV // G, p2, _splat_i(0))
        cnt = jnp.max(cntv)
        padbase = jnp.minimum(cnt, CMAX)
        for u in range(SU):
            cval[pl.ds(padbase + u * L, L)] = neg16

        def normal(_):
            _select_topk(
                val_load=lambda j: cval[pl.ds(j * L, L)],
                val_kill=lambda fsp: plsc.store_scatter(
                    cval, [fsp], neg16, mask=lane0),
                idx_of=lambda fsp: plsc.load_gather(cidx, [fsp]),
                ngroups=(cnt + SU * L - 1) // (SU * L),
                ovbuf=ovbuf, oibuf=oibuf, lane0=lane0, iota=iota)
            return 0

        def fallback(_):
            _select_topk(
                val_load=lambda j: row[pl.ds(j * L, L)],
                val_kill=lambda fsp: plsc.store_scatter(
                    row, [fsp], neg16, mask=lane0),
                idx_of=lambda fsp: fsp,
                ngroups=NV // SU,
                ovbuf=ovbuf, oibuf=oibuf, lane0=lane0, iota=iota)
            return 0

        lax.cond(cnt <= CMAX, normal, fallback, 0)

        pltpu.sync_copy(ovbuf, outv_hbm.at[base_row + r])
        pltpu.sync_copy(oibuf, outi_hbm.at[base_row + r])


@functools.lru_cache(maxsize=1)
def _topk_call():
    return functools.partial(
        pl.kernel,
        out_type=[
            jax.ShapeDtypeStruct((B, K), jnp.float32),
            jax.ShapeDtypeStruct((B, K), jnp.int32),
        ],
        mesh=plsc.VectorSubcoreMesh(core_axis_name="c", subcore_axis_name="s"),
        compiler_params=pltpu.CompilerParams(needs_layout_passes=False),
        scratch_types=[
            pltpu.VMEM((N,), jnp.float32),
            pltpu.VMEM((N,), jnp.float32),
            pltpu.VMEM((CMAX + SU * L,), jnp.float32),
            pltpu.VMEM((CMAX + SU * L,), jnp.int32),
            pltpu.VMEM((K,), jnp.float32),
            pltpu.VMEM((K,), jnp.int32),
            pltpu.SemaphoreType.DMA,
            pltpu.SemaphoreType.DMA,
        ],
    )(_topk_body)


def kernel(accumulated):
    topk_vals, topk_idx = _topk_call()(accumulated)
    return (topk_vals, topk_idx, accumulated)
